# Initial kernel scaffold; baseline (speedup 1.0000x reference)
import jax
import jax.numpy as jnp
from jax.experimental import pallas as pl


def kernel(features, W, gamma, beta, gather_idx, scatter_idx, kernel_ptr):
    raise NotImplementedError("write your pallas kernel here")


# speedup vs baseline: 5.8201x; 5.8201x over previous
import functools

import jax
import jax.numpy as jnp
from jax import lax
from jax.experimental import pallas as pl
from jax.experimental.pallas import tpu as pltpu
from jax.experimental.pallas import tpu_sc as plsc

I32 = jnp.int32
F32 = jnp.float32

T2 = 2048
CH = 128
NW = 32
STRIP = 16384
BIG = 1 << 30


def _ceil_to(a, b):
    return (a + b - 1) // b * b


def _sc_gather(features, gidx2, M1):
    N, C = features.shape
    bw = M1 // NW
    mesh = plsc.VectorSubcoreMesh(core_axis_name="c", subcore_axis_name="s")

    @functools.partial(
        pl.kernel, mesh=mesh,
        out_type=jax.ShapeDtypeStruct((M1, C), F32),
        compiler_params=pltpu.CompilerParams(use_tc_tiling_on_sc=False,
                                             needs_layout_passes=False),
        scratch_types=[
            pltpu.VMEM((CH,), I32),
            pltpu.VMEM((CH, C), F32),
            pltpu.SemaphoreType.DMA,
        ])
    def k(feat_hbm, gidx_hbm, out_hbm, idx_v, rows_v, sem):
        wid = lax.axis_index("s") * 2 + lax.axis_index("c")
        base0 = wid * bw

        def body(j, carry):
            b = base0 + j * CH
            pltpu.sync_copy(gidx_hbm.at[pl.ds(b, CH)], idx_v)
            pltpu.async_copy(feat_hbm.at[idx_v], rows_v, sem).wait()
            pltpu.sync_copy(rows_v, out_hbm.at[pl.ds(b, CH)])
            return carry

        lax.fori_loop(0, bw // CH, body, 0)

    return k(features, gidx2)


def _sc_scatter(msg, sidx2, NPAD, NSTR):
    M1, C = msg.shape
    bw = M1 // NW
    nch = bw // CH
    ACC = STRIP + 16
    ZR = ACC // 16
    WR = STRIP // 16
    mesh = plsc.VectorSubcoreMesh(core_axis_name="c", subcore_axis_name="s")

    @functools.partial(
        pl.kernel, mesh=mesh,
        out_type=jax.ShapeDtypeStruct((2 * NPAD, C), F32),
        compiler_params=pltpu.CompilerParams(use_tc_tiling_on_sc=False,
                                             needs_layout_passes=False),
        scratch_types=[
            pltpu.VMEM((bw,), I32),
            pltpu.VMEM((CH,), I32),
            pltpu.VMEM((CH, C), F32),
            pltpu.VMEM((ZR, C), F32),
            pltpu.VMEM_SHARED((ACC, C), F32),
            pltpu.SemaphoreType.DMA,
        ])
    def k(msg_hbm, sidx_hbm, out_hbm, sidxall_v, lidx_v, msg_v, zrow_v,
          acc_sh, sem):
        cidx = lax.axis_index("c")
        tid = lax.axis_index("s")
        wid = tid * 2 + cidx
        base0 = wid * bw
        pltpu.sync_copy(sidx_hbm.at[pl.ds(base0, bw)], sidxall_v)

        zero16 = jnp.zeros((16,), F32)

        def zbody(i, carry):
            zrow_v[i, pl.ds(0, 16)] = zero16
            zrow_v[i, pl.ds(16, 16)] = zero16
            return carry

        lax.fori_loop(0, ZR, zbody, 0)

        for s in range(NSTR):
            sbase = s * STRIP
            pltpu.sync_copy(zrow_v, acc_sh.at[pl.ds(tid * ZR, ZR)])
            plsc.subcore_barrier()

            lo_v = jnp.full((16,), sbase, I32)
            hi_v = jnp.full((16,), sbase + STRIP, I32)
            dum_v = jnp.full((16,), STRIP, I32) + lax.broadcast(tid, (16,))

            def cbody(j, carry):
                cb = j * CH
                nv = jnp.zeros((16,), I32)
                for g in range(8):
                    sv = sidxall_v[pl.ds(cb + g * 16, 16)]
                    in_strip = (sv >= lo_v) & (sv < hi_v)
                    li = jnp.where(in_strip, sv - lo_v, dum_v)
                    lidx_v[pl.ds(g * 16, 16)] = li
                    nv = nv + jnp.where(in_strip, jnp.full((16,), 1, I32),
                                        jnp.full((16,), 0, I32))
                any_n = jnp.sum(nv)

                @pl.when(any_n > 0)
                def _():
                    pltpu.sync_copy(msg_hbm.at[pl.ds(base0 + cb, CH)], msg_v)
                    pltpu.sync_copy(msg_v, acc_sh.at[lidx_v], add=True)

                return carry

            lax.fori_loop(0, nch, cbody, 0)
            plsc.subcore_barrier()
            pltpu.sync_copy(
                acc_sh.at[pl.ds(tid * WR, WR)],
                out_hbm.at[pl.ds(cidx * NPAD + sbase + tid * WR, WR)])
            plsc.subcore_barrier()

    return k(msg, sidx2)


def kernel(features, W, gamma, beta, gather_idx, scatter_idx, kernel_ptr):
    N, C = features.shape
    K = W.shape[0]
    M = gather_idx.shape[0]
    M1 = _ceil_to(M + K * T2, NW * CH * 2)
    M1 = _ceil_to(M1, T2)
    nt2 = M1 // T2
    NSTR = -(-N // STRIP)
    NPAD = NSTR * STRIP

    ptr = kernel_ptr.astype(I32)
    counts = ptr[1:] - ptr[:-1]
    padc = _ceil_to(counts, T2)
    pstart = jnp.concatenate(
        [jnp.zeros((1,), I32), jnp.cumsum(padc).astype(I32)])
    shift = pstart[:-1] - ptr[:-1]
    entry = jnp.arange(M, dtype=I32)
    kid = (jnp.searchsorted(ptr, entry, side='right') - 1).astype(I32)
    q = entry + shift[kid]
    gidx2 = jnp.zeros((M1,), I32).at[q].set(gather_idx.astype(I32))
    sidx2 = jnp.full((M1,), BIG, I32).at[q].set(scatter_idx.astype(I32))
    tile_k = (jnp.searchsorted(
        pstart, jnp.arange(nt2, dtype=I32) * T2, side='right') - 1).astype(I32)
    tile_k = jnp.clip(tile_k, 0, K - 1)

    gathered = _sc_gather(features, gidx2, M1)

    def gemm_body(tk_ref, x_ref, w_ref, o_ref):
        o_ref[...] = jnp.dot(x_ref[...], w_ref[0],
                             preferred_element_type=F32)

    gs = pltpu.PrefetchScalarGridSpec(
        num_scalar_prefetch=1, grid=(nt2,),
        in_specs=[pl.BlockSpec((T2, C), lambda i, tk: (i, 0)),
                  pl.BlockSpec((1, C, C), lambda i, tk: (tk[i], 0, 0))],
        out_specs=pl.BlockSpec((T2, C), lambda i, tk: (i, 0)))
    msg = pl.pallas_call(
        gemm_body, grid_spec=gs,
        out_shape=jax.ShapeDtypeStruct((M1, C), F32))(tile_k, gathered, W)

    out2 = _sc_scatter(msg, sidx2, NPAD, NSTR)
    out2 = out2.reshape(2, NPAD, C)

    nblk = NPAD // T2

    def stats_body(a_ref, b_ref, o_ref):
        @pl.when(pl.program_id(0) == 0)
        def _():
            o_ref[...] = jnp.zeros_like(o_ref)
        x = a_ref[0] + b_ref[0]
        o_ref[0, :] += jnp.sum(x, axis=0)
        o_ref[1, :] += jnp.sum(x * x, axis=0)

    sums = pl.pallas_call(
        stats_body, grid=(nblk,),
        in_specs=[pl.BlockSpec((1, T2, C), lambda i: (0, i, 0)),
                  pl.BlockSpec((1, T2, C), lambda i: (1, i, 0))],
        out_specs=pl.BlockSpec((8, C), lambda i: (0, 0)),
        out_shape=jax.ShapeDtypeStruct((8, C), F32))(out2, out2)

    mean = sums[0] / N
    var = sums[1] / N - mean * mean
    scale = gamma * lax.rsqrt(var + 1e-5)
    shift_bn = beta - mean * scale

    nblk2 = -(-N // T2)

    def norm_body(a_ref, b_ref, s_ref, t_ref, o_ref):
        x = a_ref[0] + b_ref[0]
        o_ref[...] = jnp.maximum(x * s_ref[0:1, :] + t_ref[0:1, :], 0.0)

    y = pl.pallas_call(
        norm_body, grid=(nblk2,),
        in_specs=[pl.BlockSpec((1, T2, C), lambda i: (0, i, 0)),
                  pl.BlockSpec((1, T2, C), lambda i: (1, i, 0)),
                  pl.BlockSpec((1, C), lambda i: (0, 0)),
                  pl.BlockSpec((1, C), lambda i: (0, 0))],
        out_specs=pl.BlockSpec((T2, C), lambda i: (i, 0)),
        out_shape=jax.ShapeDtypeStruct((N, C), F32))(
            out2, out2, scale.reshape(1, C), shift_bn.reshape(1, C))
    return y
